# baseline (speedup 1.0000x reference)
import functools

import jax
import jax.numpy as jnp
from jax import lax
from jax.experimental import pallas as pl
from jax.experimental.pallas import tpu as pltpu
from jax.experimental.pallas import tpu_sc as plsc

T = 512
CH = 3
H = 224
W = 224
NKEY = 171
NW = 32
NS = CH * H
SPT = NS // NW
NCH = 14
CW = W // NCH
FU = 10


def kernel(video):
    v3 = jnp.transpose(video, (1, 2, 3, 0)).reshape(NS, W, T)
    mesh = plsc.VectorSubcoreMesh(core_axis_name="c", subcore_axis_name="s")

    @functools.partial(
        pl.kernel,
        mesh=mesh,
        out_type=jax.ShapeDtypeStruct((NKEY, CH, H, W), jnp.float32),
        scratch_types=(
            [pltpu.VMEM((CW, T), jnp.float32)] * 2
            + [pltpu.VMEM((NKEY, W), jnp.float32)] * 2
            + [pltpu.SemaphoreType.DMA] * 4
        ),
        compiler_params=pltpu.CompilerParams(
            use_tc_tiling_on_sc=True, needs_layout_passes=False),
    )
    def k(v_hbm, o_hbm, ib0, ib1, ob0, ob1, *sems):
        inbufs = (ib0, ib1)
        outbufs = (ob0, ob1)
        gsems = sems[0:2]
        ssems = sems[2:4]
        wid = lax.axis_index("s") * 2 + lax.axis_index("c")
        w16 = lax.iota(jnp.int32, 16)

        def in_copy(u, k_):
            return pltpu.make_async_copy(
                v_hbm.at[u * NW + wid, pl.ds(k_ * CW, CW)],
                inbufs[k_ % 2],
                gsems[k_ % 2],
            )

        def out_copy(u, p):
            s = u * NW + wid
            return pltpu.make_async_copy(
                outbufs[p], o_hbm.at[:, s // H, s % H], ssems[p])

        def do_stripe(u, p, prefetch_next):
            for k_ in range(NCH):
                if k_ + 1 < NCH:
                    in_copy(u, k_ + 1).start()
                elif prefetch_next:
                    in_copy(u + 1, 0).start()
                if k_ == 0:
                    @pl.when(u >= 2)
                    def _():
                        out_copy(u - 2, p).wait()
                in_copy(u, k_).wait()
                inb = inbufs[k_ % 2]
                outb = outbufs[p]
                col = k_ * CW

                v = plsc.load_gather(inb, [w16, jnp.zeros((16,), jnp.int32)])
                outb[0, pl.ds(col, CW)] = v

                def fblk(b, _):
                    f0 = 1 + b * FU
                    for j in range(FU):
                        f = f0 + j
                        srcv = jnp.full((16,), 3 * f - 1, jnp.int32)
                        vj = plsc.load_gather(inb, [w16, srcv])
                        outb[f, pl.ds(col, CW)] = vj
                    return 0

                lax.fori_loop(0, (NKEY - 1) // FU, fblk, 0)
            out_copy(u, p).start()

        in_copy(0, 0).start()

        def pair(pr, _):
            u = pr * 2
            do_stripe(u, 0, True)
            do_stripe(u + 1, 1, True)
            return 0

        lax.fori_loop(0, SPT // 2, pair, 0)
        do_stripe(SPT - 1, 0, False)
        for u, p in ((SPT - 2, 1), (SPT - 1, 0)):
            out_copy(u, p).wait()

    return k(v3)


# speedup vs baseline: 1.1668x; 1.1668x over previous
import functools

import jax
import jax.numpy as jnp
from jax import lax
from jax.experimental import pallas as pl
from jax.experimental.pallas import tpu as pltpu
from jax.experimental.pallas import tpu_sc as plsc

T = 512
CH = 3
H = 224
W = 224
NKEY = 171
NW = 32
NS = CH * H
SPT = NS // NW
NCH = 14
CW = W // NCH
FB = 11
FPAD = FB * 16


def kernel(video):
    v3 = jnp.transpose(video, (1, 2, 3, 0)).reshape(NS, W, T)
    mesh = plsc.VectorSubcoreMesh(core_axis_name="c", subcore_axis_name="s")

    @functools.partial(
        pl.kernel,
        mesh=mesh,
        out_type=jax.ShapeDtypeStruct((NKEY, CH, H, W), jnp.float32),
        scratch_types=(
            [pltpu.VMEM((CW, T), jnp.float32)] * 2
            + [pltpu.VMEM((NKEY, W), jnp.float32)] * 2
            + [pltpu.SemaphoreType.DMA] * 4
        ),
        compiler_params=pltpu.CompilerParams(
            use_tc_tiling_on_sc=True, needs_layout_passes=False),
    )
    def k(v_hbm, o_hbm, ib0, ib1, ob0, ob1, *sems):
        inbufs = (ib0, ib1)
        outbufs = (ob0, ob1)
        gsems = sems[0:2]
        ssems = sems[2:4]
        wid = lax.axis_index("s") * 2 + lax.axis_index("c")
        w16 = lax.iota(jnp.int32, 16)

        def in_copy(u, k_):
            return pltpu.make_async_copy(
                v_hbm.at[u * NW + wid, pl.ds(k_ * CW, CW)],
                inbufs[k_ % 2],
                gsems[k_ % 2],
            )

        def out_copy(u, p):
            s = u * NW + wid
            return pltpu.make_async_copy(
                outbufs[p], o_hbm.at[:, s // H, s % H], ssems[p])

        def do_stripe(u, p, prefetch_next):
            for k_ in range(NCH):
                if k_ + 1 < NCH:
                    in_copy(u, k_ + 1).start()
                elif prefetch_next:
                    in_copy(u + 1, 0).start()
                if k_ == 0:
                    @pl.when(u >= 2)
                    def _():
                        out_copy(u - 2, p).wait()
                in_copy(u, k_).wait()
                inb = inbufs[k_ % 2]
                outb = outbufs[p]

                def fblk(b, _):
                    fv = b * 16 + w16
                    srcv = jnp.clip(3 * fv - 1, 0, T - 1)
                    fmask = fv < NKEY
                    fvc = jnp.minimum(fv, NKEY - 1)
                    for w_ in range(CW):
                        wv = jnp.full((16,), w_, jnp.int32)
                        cv = jnp.full((16,), k_ * CW + w_, jnp.int32)
                        vj = plsc.load_gather(inb, [wv, srcv])
                        plsc.store_scatter(outb, [fvc, cv], vj, mask=fmask)
                    return 0

                lax.fori_loop(0, FB, fblk, 0)
            out_copy(u, p).start()

        in_copy(0, 0).start()

        def pair(pr, _):
            u = pr * 2
            do_stripe(u, 0, True)
            do_stripe(u + 1, 1, True)
            return 0

        lax.fori_loop(0, SPT // 2, pair, 0)
        do_stripe(SPT - 1, 0, False)
        for u, p in ((SPT - 2, 1), (SPT - 1, 0)):
            out_copy(u, p).wait()

    return k(v3)


# speedup vs baseline: 1.4484x; 1.2414x over previous
import functools

import jax
import jax.numpy as jnp
from jax import lax
from jax.experimental import pallas as pl
from jax.experimental.pallas import tpu as pltpu
from jax.experimental.pallas import tpu_sc as plsc

T = 512
CH = 3
H = 224
W = 224
NKEY = 171
NW = 32
NS = CH * H
SPT = NS // NW
NCH = 14
CW = W // NCH
FB = 11
FPAD = FB * 16


def kernel(video):
    v3 = jnp.transpose(video, (1, 2, 3, 0)).reshape(NS, W, T)
    mesh = plsc.VectorSubcoreMesh(core_axis_name="c", subcore_axis_name="s")

    @functools.partial(
        pl.kernel,
        mesh=mesh,
        out_type=jax.ShapeDtypeStruct((NKEY, CH, H, W), jnp.float32),
        scratch_types=(
            [pltpu.VMEM((CW, T), jnp.float32)] * 2
            + [pltpu.VMEM((NKEY, W), jnp.float32)] * 2
            + [pltpu.SemaphoreType.DMA] * 4
        ),
        compiler_params=pltpu.CompilerParams(
            use_tc_tiling_on_sc=True, needs_layout_passes=False),
    )
    def k(v_hbm, o_hbm, ib0, ib1, ob0, ob1, *sems):
        inbufs = (ib0, ib1)
        outbufs = (ob0, ob1)
        gsems = sems[0:2]
        ssems = sems[2:4]
        wid = lax.axis_index("s") * 2 + lax.axis_index("c")
        w16 = lax.iota(jnp.int32, 16)

        def in_copy(u, k_):
            return pltpu.make_async_copy(
                v_hbm.at[u * NW + wid, pl.ds(k_ * CW, CW)],
                inbufs[k_ % 2],
                gsems[k_ % 2],
            )

        def out_copy(u, p):
            s = u * NW + wid
            return pltpu.make_async_copy(
                outbufs[p], o_hbm.at[:, s // H, s % H], ssems[p])

        def do_stripe(u, p, prefetch_next):
            for k_ in range(NCH):
                if k_ + 1 < NCH:
                    in_copy(u, k_ + 1).start()
                elif prefetch_next:
                    in_copy(u + 1, 0).start()
                if k_ == 0:
                    @pl.when(u >= 2)
                    def _():
                        out_copy(u - 2, p).wait()
                in_copy(u, k_).wait()
                inb = inbufs[k_ % 2]
                outb = outbufs[p]

                def fblk(b, _):
                    fv = b * 16 + w16
                    srcv = jnp.clip(3 * fv - 1, 0, T - 1)
                    fmask = fv < NKEY
                    fvc = jnp.minimum(fv, NKEY - 1)
                    for w0 in range(0, CW, 4):
                        vs = []
                        for d in range(4):
                            wv = jnp.full((16,), w0 + d, jnp.int32)
                            vs.append(plsc.load_gather(inb, [wv, srcv]))
                        for d in range(4):
                            cv = jnp.full((16,), k_ * CW + w0 + d, jnp.int32)
                            plsc.store_scatter(
                                outb, [fvc, cv], vs[d], mask=fmask)
                    return 0

                lax.fori_loop(0, FB, fblk, 0)
            out_copy(u, p).start()

        in_copy(0, 0).start()

        def pair(pr, _):
            u = pr * 2
            do_stripe(u, 0, True)
            do_stripe(u + 1, 1, True)
            return 0

        lax.fori_loop(0, SPT // 2, pair, 0)
        do_stripe(SPT - 1, 0, False)
        for u, p in ((SPT - 2, 1), (SPT - 1, 0)):
            out_copy(u, p).wait()

    return k(v3)
